# baseline (speedup 1.0000x reference)
import functools

import jax
import jax.numpy as jnp
from jax import lax
from jax.experimental import pallas as pl
from jax.experimental.pallas import tpu as pltpu
from jax.experimental.pallas import tpu_sc as plsc

EMBED = 64
IDX_CHUNK = 128


def _sc_gather(dom_tab, sys_tab, dom_idx2d, sys_idx2d, batch):
    info = plsc.get_sparse_core_info()
    nw = info.num_cores * info.num_subcores
    bpw = batch // nw
    nch = bpw // IDX_CHUNK

    mesh = plsc.VectorSubcoreMesh(core_axis_name="c", subcore_axis_name="s")

    @functools.partial(
        pl.kernel,
        mesh=mesh,
        compiler_params=pltpu.CompilerParams(use_tc_tiling_on_sc=False),
        out_type=(
            jax.ShapeDtypeStruct((batch, EMBED), jnp.float32),
            jax.ShapeDtypeStruct((batch, EMBED), jnp.float32),
        ),
        scratch_types=[
            pltpu.VMEM((nch, IDX_CHUNK), jnp.int32),
            pltpu.VMEM((nch, IDX_CHUNK), jnp.int32),
            pltpu.VMEM((bpw, EMBED), jnp.float32),
            pltpu.VMEM((bpw, EMBED), jnp.float32),
            pltpu.SemaphoreType.DMA,
            pltpu.SemaphoreType.DMA((4,)),
            pltpu.SemaphoreType.DMA,
        ],
    )
    def gather(dom_tab_h, sys_tab_h, dom_idx_h, sys_idx_h,
               dom_out_h, sys_out_h, didx_v, sidx_v, drows_v, srows_v,
               sem, gsems, wsem):
        wid = lax.axis_index("s") * info.num_cores + lax.axis_index("c")
        base = wid * bpw
        row0 = wid * nch
        i1 = pltpu.async_copy(dom_idx_h.at[pl.ds(row0, nch)], didx_v, sem)
        i2 = pltpu.async_copy(sys_idx_h.at[pl.ds(row0, nch)], sidx_v, sem)
        i1.wait()
        i2.wait()
        gathers = []
        for j in range(nch):
            dst = pl.ds(j * IDX_CHUNK, IDX_CHUNK)
            gathers.append((
                pltpu.async_copy(dom_tab_h.at[didx_v.at[j]], drows_v.at[dst],
                                 gsems.at[j]),
                pltpu.async_copy(sys_tab_h.at[sidx_v.at[j]], srows_v.at[dst],
                                 gsems.at[j])))
        writes = []
        for j in range(nch):
            gd, gs = gathers[j]
            gd.wait()
            gs.wait()
            dst = pl.ds(j * IDX_CHUNK, IDX_CHUNK)
            out = pl.ds(base + j * IDX_CHUNK, IDX_CHUNK)
            writes.append(
                pltpu.async_copy(drows_v.at[dst], dom_out_h.at[out], wsem))
            writes.append(
                pltpu.async_copy(srows_v.at[dst], sys_out_h.at[out], wsem))
        for w in writes:
            w.wait()

    return gather(dom_tab, sys_tab, dom_idx2d, sys_idx2d)


def _mlp_body(dom_ref, sys_ref, w1_ref, b1_ref, g_ref, bt_ref,
              w2_ref, b2_ref, out_ref):
    h = jnp.dot(dom_ref[...], w1_ref[:EMBED, :], preferred_element_type=jnp.float32)
    h = h + jnp.dot(sys_ref[...], w1_ref[EMBED:, :], preferred_element_type=jnp.float32)
    h = h + b1_ref[...]
    mean = jnp.mean(h, axis=1, keepdims=True)
    var = jnp.mean((h - mean) * (h - mean), axis=1, keepdims=True)
    h = (h - mean) * lax.rsqrt(var + 1e-5) * g_ref[...] + bt_ref[...]
    h = h * jax.nn.sigmoid(h)
    out_ref[...] = (
        jnp.dot(h, w2_ref[...], preferred_element_type=jnp.float32) + b2_ref[...])


def _tc_mlp(dom_emb, sys_emb, W1, b1, ln_gamma, ln_beta, W2, b2):
    batch = dom_emb.shape[0]
    d2 = 2 * EMBED
    blk = min(batch, 4096)
    grid = (batch // blk,)
    full = lambda r, c: pl.BlockSpec((r, c), lambda i: (0, 0))
    return pl.pallas_call(
        _mlp_body,
        grid=grid,
        in_specs=[
            pl.BlockSpec((blk, EMBED), lambda i: (i, 0)),
            pl.BlockSpec((blk, EMBED), lambda i: (i, 0)),
            full(d2, d2),
            full(1, d2),
            full(1, d2),
            full(1, d2),
            full(d2, EMBED),
            full(1, EMBED),
        ],
        out_specs=pl.BlockSpec((blk, EMBED), lambda i: (i, 0)),
        out_shape=jax.ShapeDtypeStruct((batch, EMBED), jnp.float32),
    )(dom_emb, sys_emb, W1, b1[None], ln_gamma[None], ln_beta[None],
      W2, b2[None])


def kernel(domain_ids, system_ids, domain_table, system_table,
           W1, b1, ln_gamma, ln_beta, W2, b2):
    batch = domain_ids.shape[0]
    dom_idx2d = domain_ids.astype(jnp.int32).reshape(-1, IDX_CHUNK)
    sys_idx2d = system_ids.astype(jnp.int32).reshape(-1, IDX_CHUNK)
    dom_emb, sys_emb = _sc_gather(domain_table, system_table,
                                  dom_idx2d, sys_idx2d, batch)
    return _tc_mlp(dom_emb, sys_emb, W1, b1, ln_gamma, ln_beta, W2, b2)


# speedup vs baseline: 1.1339x; 1.1339x over previous
import functools

import jax
import jax.numpy as jnp
from jax import lax
from jax.experimental import pallas as pl
from jax.experimental.pallas import tpu as pltpu
from jax.experimental.pallas import tpu_sc as plsc

EMBED = 64
IDX_CHUNK = 128
LANES = 16


def _combo_body(dom_ref, sys_ref, w1_ref, b1_ref, g_ref, bt_ref,
                w2_ref, b2_ref, out_ref):
    hd = jnp.dot(dom_ref[0], w1_ref[:EMBED, :],
                 preferred_element_type=jnp.float32)
    hs = jnp.dot(sys_ref[...], w1_ref[EMBED:, :],
                 preferred_element_type=jnp.float32)
    h = hd + hs + b1_ref[...]
    mean = jnp.mean(h, axis=1, keepdims=True)
    var = jnp.mean((h - mean) * (h - mean), axis=1, keepdims=True)
    h = (h - mean) * lax.rsqrt(var + 1e-5) * g_ref[...] + bt_ref[...]
    h = h * jax.nn.sigmoid(h)
    out_ref[...] = (jnp.dot(h, w2_ref[...],
                            preferred_element_type=jnp.float32)
                    + b2_ref[...])[None]


def _tc_combo(dom_tab, sys_tab, W1, b1, ln_gamma, ln_beta, W2, b2):
    n = dom_tab.shape[0]
    d2 = 2 * EMBED
    full = lambda r, c: pl.BlockSpec((r, c), lambda i: (0, 0))
    return pl.pallas_call(
        _combo_body,
        grid=(n,),
        in_specs=[
            pl.BlockSpec((1, 1, EMBED), lambda i: (i, 0, 0)),
            pl.BlockSpec((n, EMBED), lambda i: (0, 0)),
            full(d2, d2),
            full(1, d2),
            full(1, d2),
            full(1, d2),
            full(d2, EMBED),
            full(1, EMBED),
        ],
        out_specs=pl.BlockSpec((1, n, EMBED), lambda i: (i, 0, 0)),
        out_shape=jax.ShapeDtypeStruct((n, n, EMBED), jnp.float32),
    )(dom_tab[:, None], sys_tab, W1, b1[None], ln_gamma[None], ln_beta[None],
      W2, b2[None])


def _sc_gather_combo(combo_tab, dom_idx2d, sys_idx2d, batch, n):
    info = plsc.get_sparse_core_info()
    nw = info.num_cores * info.num_subcores
    bpw = batch // nw
    nch = bpw // IDX_CHUNK

    mesh = plsc.VectorSubcoreMesh(core_axis_name="c", subcore_axis_name="s")

    @functools.partial(
        pl.kernel,
        mesh=mesh,
        compiler_params=pltpu.CompilerParams(use_tc_tiling_on_sc=False),
        out_type=jax.ShapeDtypeStruct((batch, EMBED), jnp.float32),
        scratch_types=[
            pltpu.VMEM((nch, IDX_CHUNK), jnp.int32),
            pltpu.VMEM((nch, IDX_CHUNK), jnp.int32),
            pltpu.VMEM((nch, IDX_CHUNK), jnp.int32),
            pltpu.VMEM((bpw, EMBED), jnp.float32),
            pltpu.SemaphoreType.DMA,
            pltpu.SemaphoreType.DMA,
        ],
    )
    def gather(combo_h, dom_idx_h, sys_idx_h, out_h,
               didx_v, sidx_v, cidx_v, rows_v, sem, wsem):
        wid = lax.axis_index("s") * info.num_cores + lax.axis_index("c")
        base = wid * bpw
        row0 = wid * nch
        i1 = pltpu.async_copy(dom_idx_h.at[pl.ds(row0, nch)], didx_v, sem)
        i2 = pltpu.async_copy(sys_idx_h.at[pl.ds(row0, nch)], sidx_v, sem)
        i1.wait()
        i2.wait()
        for j in range(nch):
            for k in range(IDX_CHUNK // LANES):
                sl = pl.ds(k * LANES, LANES)
                cidx_v[j, sl] = didx_v[j, sl] * n + sidx_v[j, sl]
        copies = []
        for j in range(nch):
            dst = pl.ds(j * IDX_CHUNK, IDX_CHUNK)
            copies.append(
                pltpu.async_copy(combo_h.at[cidx_v.at[j]], rows_v.at[dst], sem))
        for c in copies:
            c.wait()
        pltpu.async_copy(rows_v, out_h.at[pl.ds(base, bpw)], wsem).wait()

    return gather(combo_tab, dom_idx2d, sys_idx2d)


def kernel(domain_ids, system_ids, domain_table, system_table,
           W1, b1, ln_gamma, ln_beta, W2, b2):
    batch = domain_ids.shape[0]
    n = domain_table.shape[0]
    dom_idx2d = domain_ids.astype(jnp.int32).reshape(-1, IDX_CHUNK)
    sys_idx2d = system_ids.astype(jnp.int32).reshape(-1, IDX_CHUNK)
    combo = _tc_combo(domain_table, system_table, W1, b1,
                      ln_gamma, ln_beta, W2, b2).reshape(n * n, EMBED)
    return _sc_gather_combo(combo, dom_idx2d, sys_idx2d, batch, n)


# speedup vs baseline: 1.8183x; 1.6036x over previous
import functools

import jax
import jax.numpy as jnp
from jax import lax
from jax.experimental import pallas as pl
from jax.experimental.pallas import tpu as pltpu
from jax.experimental.pallas import tpu_sc as plsc

EMBED = 64
IDX_CHUNK = 128
LANES = 16


def _combo_body(dom_ref, sys_ref, w1_ref, b1_ref, g_ref, bt_ref,
                w2_ref, b2_ref, out_ref):
    n = dom_ref.shape[0]
    hd = jnp.dot(dom_ref[...], w1_ref[:EMBED, :],
                 preferred_element_type=jnp.float32)
    hs = jnp.dot(sys_ref[...], w1_ref[EMBED:, :],
                 preferred_element_type=jnp.float32)
    h = hd[:, None, :] + hs[None, :, :] + b1_ref[...]
    mean = jnp.mean(h, axis=-1, keepdims=True)
    var = jnp.mean((h - mean) * (h - mean), axis=-1, keepdims=True)
    h = (h - mean) * lax.rsqrt(var + 1e-5) * g_ref[...] + bt_ref[...]
    h = h * jax.nn.sigmoid(h)
    h2 = h.reshape(n * n, 2 * EMBED)
    out_ref[...] = (jnp.dot(h2, w2_ref[...],
                            preferred_element_type=jnp.float32)
                    + b2_ref[...])


def _tc_combo(dom_tab, sys_tab, W1, b1, ln_gamma, ln_beta, W2, b2):
    n = dom_tab.shape[0]
    d2 = 2 * EMBED
    full = lambda r, c: pl.BlockSpec((r, c), lambda: (0, 0))
    return pl.pallas_call(
        _combo_body,
        in_specs=[
            full(n, EMBED),
            full(n, EMBED),
            full(d2, d2),
            full(1, d2),
            full(1, d2),
            full(1, d2),
            full(d2, EMBED),
            full(1, EMBED),
        ],
        out_specs=pl.BlockSpec((n * n, EMBED), lambda: (0, 0)),
        out_shape=jax.ShapeDtypeStruct((n * n, EMBED), jnp.float32),
    )(dom_tab, sys_tab, W1, b1[None], ln_gamma[None], ln_beta[None],
      W2, b2[None])


def _sc_gather_combo(combo_tab, dom_idx2d, sys_idx2d, batch, n):
    info = plsc.get_sparse_core_info()
    nw = info.num_cores * info.num_subcores
    bpw = batch // nw
    nch = bpw // IDX_CHUNK

    mesh = plsc.VectorSubcoreMesh(core_axis_name="c", subcore_axis_name="s")

    @functools.partial(
        pl.kernel,
        mesh=mesh,
        compiler_params=pltpu.CompilerParams(use_tc_tiling_on_sc=False),
        out_type=jax.ShapeDtypeStruct((batch, EMBED), jnp.float32),
        scratch_types=[
            pltpu.VMEM((nch, IDX_CHUNK), jnp.int32),
            pltpu.VMEM((nch, IDX_CHUNK), jnp.int32),
            pltpu.VMEM((nch, IDX_CHUNK), jnp.int32),
            pltpu.VMEM((bpw, EMBED), jnp.float32),
            pltpu.SemaphoreType.DMA,
            pltpu.SemaphoreType.DMA,
        ],
    )
    def gather(combo_h, dom_idx_h, sys_idx_h, out_h,
               didx_v, sidx_v, cidx_v, rows_v, sem, wsem):
        wid = lax.axis_index("s") * info.num_cores + lax.axis_index("c")
        base = wid * bpw
        row0 = wid * nch
        i1 = pltpu.async_copy(dom_idx_h.at[pl.ds(row0, nch)], didx_v, sem)
        i2 = pltpu.async_copy(sys_idx_h.at[pl.ds(row0, nch)], sidx_v, sem)
        i1.wait()
        i2.wait()
        for j in range(nch):
            for k in range(IDX_CHUNK // LANES):
                sl = pl.ds(k * LANES, LANES)
                cidx_v[j, sl] = didx_v[j, sl] * n + sidx_v[j, sl]
        copies = []
        for j in range(nch):
            dst = pl.ds(j * IDX_CHUNK, IDX_CHUNK)
            copies.append(
                pltpu.async_copy(combo_h.at[cidx_v.at[j]], rows_v.at[dst], sem))
        for c in copies:
            c.wait()
        pltpu.async_copy(rows_v, out_h.at[pl.ds(base, bpw)], wsem).wait()

    return gather(combo_tab, dom_idx2d, sys_idx2d)


def kernel(domain_ids, system_ids, domain_table, system_table,
           W1, b1, ln_gamma, ln_beta, W2, b2):
    batch = domain_ids.shape[0]
    n = domain_table.shape[0]
    dom_idx2d = domain_ids.astype(jnp.int32).reshape(-1, IDX_CHUNK)
    sys_idx2d = system_ids.astype(jnp.int32).reshape(-1, IDX_CHUNK)
    combo = _tc_combo(domain_table, system_table, W1, b1,
                      ln_gamma, ln_beta, W2, b2)
    return _sc_gather_combo(combo, dom_idx2d, sys_idx2d, batch, n)
